# Initial kernel scaffold; baseline (speedup 1.0000x reference)
import jax
import jax.numpy as jnp
from jax.experimental import pallas as pl


def kernel(positions, token_content, temporal_state, mem_keys, mem_values, W_pos, b_pos, W_content, b_content, similarity_weight, W_gate, b_gate, W_evol, b_evol):
    raise NotImplementedError("write your pallas kernel here")


# speedup vs baseline: 3.0312x; 3.0312x over previous
import functools

import jax
import jax.numpy as jnp
from jax import lax
from jax.experimental import pallas as pl
from jax.experimental.pallas import tpu as pltpu
from jax.experimental.pallas import tpu_sc as plsc

Q = 1024
K = 131072
D = 128
TOP_K = 3
BK = 2048
PAD = 8


def _topk_body(tc_ref, wc_ref, bc_ref, mk_ref, vals_ref, idx_ref, ck_ref):
    k = pl.program_id(0)

    @pl.when(k == 0)
    def _init():
        ck = lax.dot_general(tc_ref[...], wc_ref[...], (((1,), (1,)), ((), ())),
                             preferred_element_type=jnp.float32)
        ck_ref[...] = (ck + bc_ref[...]).astype(jnp.bfloat16)
        vals_ref[...] = jnp.full((Q, PAD), -jnp.inf, jnp.float32)
        idx_ref[...] = jnp.zeros((Q, PAD), jnp.int32)

    mk = mk_ref[...].astype(jnp.bfloat16)
    s = lax.dot_general(ck_ref[...], mk, (((1,), (1,)), ((), ())),
                        preferred_element_type=jnp.float32)

    iota = lax.broadcasted_iota(jnp.int32, (Q, BK), 1)
    neg_inf = jnp.float32(-jnp.inf)
    bv, bi = [], []
    for _ in range(TOP_K):
        m = jnp.max(s, axis=1, keepdims=True)
        a = jnp.min(jnp.where(s == m, iota, BK), axis=1, keepdims=True)
        bv.append(m)
        bi.append(a + k * BK)
        s = jnp.where(iota == a, neg_inf, s)

    rv, ri = vals_ref[...], idx_ref[...]
    a1v, a2v, a3v = rv[:, 0:1], rv[:, 1:2], rv[:, 2:3]
    a1i, a2i, a3i = ri[:, 0:1], ri[:, 1:2], ri[:, 2:3]
    b1v, b2v, b3v = bv
    b1i, b2i, b3i = bi

    g1 = a1v >= b1v
    o1v = jnp.where(g1, a1v, b1v)
    o1i = jnp.where(g1, a1i, b1i)
    pav = jnp.where(g1, a2v, a1v)
    pai = jnp.where(g1, a2i, a1i)
    pav2 = jnp.where(g1, a3v, a2v)
    pai2 = jnp.where(g1, a3i, a2i)
    pbv = jnp.where(g1, b1v, b2v)
    pbi = jnp.where(g1, b1i, b2i)
    pbv2 = jnp.where(g1, b2v, b3v)
    pbi2 = jnp.where(g1, b2i, b3i)

    g2 = pav >= pbv
    o2v = jnp.where(g2, pav, pbv)
    o2i = jnp.where(g2, pai, pbi)
    qav = jnp.where(g2, pav2, pav)
    qai = jnp.where(g2, pai2, pai)
    qbv = jnp.where(g2, pbv, pbv2)
    qbi = jnp.where(g2, pbi, pbi2)

    g3 = qav >= qbv
    o3v = jnp.where(g3, qav, qbv)
    o3i = jnp.where(g3, qai, qbi)

    pad_v = jnp.full((Q, PAD - TOP_K), neg_inf, jnp.float32)
    pad_i = jnp.zeros((Q, PAD - TOP_K), jnp.int32)
    vals_ref[...] = jnp.concatenate([o1v, o2v, o3v, pad_v], axis=1)
    idx_ref[...] = jnp.concatenate([o1i, o2i, o3i, pad_i], axis=1)


def _topk_call(token_content, W_content, b_content_row, mem_keys):
    return pl.pallas_call(
        _topk_body,
        grid=(K // BK,),
        in_specs=[
            pl.BlockSpec((Q, D), lambda k: (0, 0)),
            pl.BlockSpec((D, D), lambda k: (0, 0)),
            pl.BlockSpec((1, D), lambda k: (0, 0)),
            pl.BlockSpec((BK, D), lambda k: (k, 0)),
        ],
        out_specs=[
            pl.BlockSpec((Q, PAD), lambda k: (0, 0)),
            pl.BlockSpec((Q, PAD), lambda k: (0, 0)),
        ],
        out_shape=[
            jax.ShapeDtypeStruct((Q, PAD), jnp.float32),
            jax.ShapeDtypeStruct((Q, PAD), jnp.int32),
        ],
        scratch_shapes=[pltpu.VMEM((Q, D), jnp.bfloat16)],
        compiler_params=pltpu.CompilerParams(
            dimension_semantics=("arbitrary",)),
    )(token_content, W_content, b_content_row, mem_keys)


def _gather_call(flat_idx, table):
    B = flat_idx.shape[0]
    info = plsc.get_sparse_core_info()
    nc, ns = info.num_cores, info.num_subcores
    nw = nc * ns
    b_per_w = B // nw
    mesh = plsc.VectorSubcoreMesh(core_axis_name="c", subcore_axis_name="s")

    @functools.partial(
        pl.kernel, mesh=mesh,
        out_type=jax.ShapeDtypeStruct((B, D), jnp.float32),
        scratch_types=[
            pltpu.VMEM((b_per_w,), jnp.int32),
            pltpu.VMEM((b_per_w, D), jnp.float32),
            pltpu.SemaphoreType.DMA,
        ],
    )
    def gather_k(idx_hbm, table_hbm, out_hbm, idx_v, rows_v, sem):
        wid = lax.axis_index("s") * nc + lax.axis_index("c")
        base = wid * b_per_w
        pltpu.sync_copy(idx_hbm.at[pl.ds(base, b_per_w)], idx_v)
        pltpu.async_copy(table_hbm.at[idx_v], rows_v, sem).wait()
        pltpu.sync_copy(rows_v, out_hbm.at[pl.ds(base, b_per_w)])

    return gather_k(flat_idx, table)


def _epilogue_body(posf_ref, wpos_ref, bpos_ref, vals_ref, gath_ref, ts_ref,
                   sw_ref, wg_ref, bg_ref, we_ref, be_ref, out_ref):
    v = vals_ref[...]
    m = v[:, 0:1]
    e = jnp.exp(v - m)
    attn = e / jnp.sum(e, axis=1, keepdims=True)
    g = gath_ref[...]
    sim = (attn[:, 0:1] * g[:, 0:D]
           + attn[:, 1:2] * g[:, D:2 * D]
           + attn[:, 2:3] * g[:, 2 * D:3 * D])
    base = posf_ref[...] * wpos_ref[...] + bpos_ref[...]
    fe = base + sw_ref[...] * sim
    gate_in = jnp.concatenate([fe, ts_ref[...]], axis=1)
    z = lax.dot_general(gate_in, wg_ref[...], (((1,), (1,)), ((), ())),
                        preferred_element_type=jnp.float32) + bg_ref[...]
    ti = jax.nn.sigmoid(z)
    ev = lax.dot_general(fe, we_ref[...], (((1,), (1,)), ((), ())),
                         preferred_element_type=jnp.float32) + be_ref[...]
    out_ref[...] = fe + ti * ev


def _epilogue_call(pos_f, wpos_row, bpos_row, top_vals, gathered,
                   temporal_state, sw, W_gate, bg_row, W_evol, be_row):
    return pl.pallas_call(
        _epilogue_body,
        out_shape=jax.ShapeDtypeStruct((Q, D), jnp.float32),
    )(pos_f, wpos_row, bpos_row, top_vals, gathered, temporal_state,
      sw, W_gate, bg_row, W_evol, be_row)


def kernel(positions, token_content, temporal_state, mem_keys, mem_values,
           W_pos, b_pos, W_content, b_content, similarity_weight,
           W_gate, b_gate, W_evol, b_evol):
    top_vals, top_idx = _topk_call(
        token_content, W_content, b_content.reshape(1, D), mem_keys)

    flat_idx = top_idx[:, :TOP_K].reshape(-1)
    gathered = _gather_call(flat_idx, mem_values)
    gathered = gathered.reshape(Q, TOP_K * D)

    pos_f = positions.astype(jnp.float32).reshape(Q, 1)
    return _epilogue_call(
        pos_f,
        W_pos.reshape(1, D),
        b_pos.reshape(1, D),
        top_vals,
        gathered,
        temporal_state,
        similarity_weight.reshape(1, 1),
        W_gate,
        b_gate.reshape(1, D),
        W_evol,
        b_evol.reshape(1, D),
    )
